# baseline (speedup 1.0000x reference)
import jax
import jax.numpy as jnp
from jax.experimental import pallas as pl
from jax.experimental.pallas import tpu as pltpu
from jax.experimental.pallas import tpu_sc as plsc

_VOCAB = 1000000
_EMB = 64
_HID = 128
_B = 4096
_L = 50
_N = _B * _L

_GATHER_WINDOW = 128
_M_BLK = 2048


def _sc_gather(table2, idx_flat):
    mesh = plsc.VectorSubcoreMesh(core_axis_name="core", subcore_axis_name="subcore")

    @pl.kernel(
        out_type=jax.ShapeDtypeStruct((_N, 2 * _EMB), table2.dtype),
        mesh=mesh,
    )
    def gather_kernel(tab_hbm, i_hbm, o_hbm):
        def body(i_vmem, o_vmem):
            pltpu.sync_copy(tab_hbm.at[i_vmem.at[0]], o_vmem)

        pltpu.emit_pipeline(
            body,
            grid=(_N // _GATHER_WINDOW,),
            in_specs=[pl.BlockSpec((1, _GATHER_WINDOW), index_map=lambda i: (0, i))],
            out_specs=[pl.BlockSpec((_GATHER_WINDOW, 2 * _EMB), index_map=lambda i: (i, 0))],
            core_axis_name=("core", "subcore"),
            dimension_semantics=(pltpu.PARALLEL,),
        )(i_hbm, o_hbm)

    return gather_kernel(table2, idx_flat)


def _tc_dense(g, parity, W, b2d):
    def body(g_ref, p_ref, w_ref, b_ref, o_ref):
        gv = g_ref[...]
        e = jnp.where(p_ref[...] == 1, gv[:, _EMB:], gv[:, :_EMB])
        h = jnp.tanh(e)
        acc = jnp.dot(h, w_ref[...], preferred_element_type=jnp.float32,
                      precision=jax.lax.Precision.HIGHEST)
        o_ref[...] = jnp.tanh(acc + b_ref[...])

    return pl.pallas_call(
        body,
        grid=(_N // _M_BLK,),
        in_specs=[
            pl.BlockSpec((_M_BLK, 2 * _EMB), lambda i: (i, 0)),
            pl.BlockSpec((_M_BLK, 1), lambda i: (i, 0)),
            pl.BlockSpec((_EMB, _HID), lambda i: (0, 0)),
            pl.BlockSpec((1, _HID), lambda i: (0, 0)),
        ],
        out_specs=pl.BlockSpec((_M_BLK, _HID), lambda i: (i, 0)),
        out_shape=jax.ShapeDtypeStruct((_N, _HID), jnp.float32),
    )(g, parity, W, b2d)


def kernel(x, table, W, b):
    xf = x.reshape(_N)
    idx2 = (xf >> 1).reshape(1, _N)
    parity = (xf & 1).reshape(_N, 1)
    table2 = table.reshape(_VOCAB // 2, 2 * _EMB)
    g = _sc_gather(table2, idx2)
    h = _tc_dense(g, parity, W, b.reshape(1, _HID))
    return h.reshape(_B, _L, _HID)
